# baseline (speedup 1.0000x reference)
import math

import jax
import jax.numpy as jnp
from jax import lax
from jax.experimental import pallas as pl
from jax.experimental.pallas import tpu as pltpu

_NB = 32
_NOUT = 3 * _NB + 1
_BOUND = 10.0
_MIN_W = 1e-05
_MIN_H = 1e-05
_MIN_D = 1e-05
_L2PI = 0.5 * math.log(2.0 * math.pi)
_PART = 16
_ROWS = 4096


def _flow_block(c_hbm, w_ref, b_ref, x_ref, s_ref, o_ref, cbuf, sem):
    rows = _ROWS
    i = pl.program_id(0)
    ngrid = pl.num_programs(0)
    c2 = c_hbm.reshape(c_hbm.shape[0] * c_hbm.shape[1], c_hbm.shape[2])

    def _start(slot, blk):
        pltpu.make_async_copy(
            c2.at[pl.ds(blk * _ROWS, _ROWS), :], cbuf.at[slot], sem.at[slot],
        ).start()

    @pl.when(i == 0)
    def _():
        _start(0, 0)

    @pl.when(i + 1 < ngrid)
    def _():
        _start((i + 1) % 2, i + 1)

    pltpu.make_async_copy(
        c2.at[pl.ds(i * _ROWS, _ROWS), :], cbuf.at[i % 2], sem.at[i % 2],
    ).wait()

    theta = lax.dot_general(
        w_ref[...], cbuf[i % 2],
        dimension_numbers=(((0,), (1,)), ((), ())),
        preferred_element_type=jnp.float32,
    ) + b_ref[...]

    uw = theta[0:_NB, :]
    uh = theta[_NB:2 * _NB, :]
    ud = theta[2 * _NB:_NOUT, :]

    xrow = x_ref[0]
    inside = (xrow >= -_BOUND) & (xrow <= _BOUND)
    xq = jnp.clip(xrow, -_BOUND, _BOUND)

    rid = lax.broadcasted_iota(jnp.int32, (_NB, rows), 0)

    def edges(u, min_v):
        e = jnp.exp(u)
        ecum = e
        for k in (1, 2, 4, 8, 16):
            ecum = ecum + jnp.concatenate(
                [jnp.zeros((k, rows), jnp.float32), ecum[:_NB - k, :]], axis=0)
        scale = (2.0 * _BOUND) * (1.0 - min_v * _NB) / ecum[_NB - 1:_NB, :]
        base = (2.0 * _BOUND * min_v) * (rid + 1).astype(jnp.float32) - _BOUND
        cum = base + ecum * scale
        cum = jnp.where(rid == _NB - 1, _BOUND, cum)
        left = jnp.concatenate(
            [jnp.full((1, rows), -_BOUND, jnp.float32), cum[:_NB - 1, :]], axis=0)
        return left, cum - left, cum

    cwl, wb, cwr = edges(uw, _MIN_W)
    chl, hb, _ = edges(uh, _MIN_H)

    one = jnp.ones((1, rows), jnp.float32)
    d = jnp.concatenate(
        [one, _MIN_D + jax.nn.softplus(ud[1:_NB, :]), one], axis=0)

    g = (xq >= cwr[:_NB - 1, :]).astype(jnp.float32)
    onehot = (jnp.concatenate([jnp.ones((1, rows), jnp.float32), g], axis=0)
              - jnp.concatenate([g, jnp.zeros((1, rows), jnp.float32)], axis=0))

    def pick(a):
        return jnp.sum(onehot * a, axis=0, keepdims=True)

    in_cw = pick(cwl)
    in_w = pick(wb)
    in_ch = pick(chl)
    in_h = pick(hb)
    d0 = pick(d[0:_NB, :])
    d1 = pick(d[1:_NB + 1, :])

    t = (xq - in_cw) / in_w
    tm = t * (1.0 - t)
    delta = in_h / in_w
    num = in_h * (delta * t * t + d0 * tm)
    den = delta + (d0 + d1 - 2.0 * delta) * tm
    outv = in_ch + num / den
    dnum = (delta * delta) * (d1 * t * t + 2.0 * delta * tm
                              + d0 * (1.0 - t) * (1.0 - t))
    lad = jnp.log(dnum) - 2.0 * jnp.log(den)

    z = jnp.where(inside, outv, xrow)
    jac = jnp.where(inside, lad, 0.0)
    prob = -_L2PI - 0.5 * z * z + jac

    p_hi = prob.astype(jnp.bfloat16)
    p_lo = (prob - p_hi.astype(jnp.float32)).astype(jnp.bfloat16)
    p2 = jnp.concatenate([p_hi, p_lo], axis=0)
    dn = (((1,), (0,)), ((), ()))
    ps2 = lax.dot_general(p2, s_ref[...], dn, preferred_element_type=jnp.float32)
    psum = ps2[0:1, :] + ps2[1:2, :]
    o_ref[...] = psum.reshape(1, 1, rows // _PART)


def kernel(x, c, W, b):
    nb, npart, _ = x.shape
    n = nb * npart
    grid = n // _ROWS
    bb = _ROWS // _PART

    be = _ROWS // npart
    b2 = b.reshape(-1, 1)
    seg = (jnp.arange(_ROWS, dtype=jnp.int32)[:, None] // _PART
           == jnp.arange(bb, dtype=jnp.int32)[None, :]).astype(jnp.bfloat16)

    out = pl.pallas_call(
        _flow_block,
        grid=(grid,),
        in_specs=[
            pl.BlockSpec(memory_space=pltpu.MemorySpace.HBM),
            pl.BlockSpec(W.shape, lambda i: (0, 0)),
            pl.BlockSpec(b2.shape, lambda i: (0, 0)),
            pl.BlockSpec((1, 1, _ROWS), lambda i: (i, 0, 0)),
            pl.BlockSpec((_ROWS, bb), lambda i: (0, 0)),
        ],
        out_specs=pl.BlockSpec((1, 1, bb), lambda i: (i, 0, 0)),
        out_shape=jax.ShapeDtypeStruct((grid, 1, bb), jnp.float32),
        scratch_shapes=[
            pltpu.VMEM((2, _ROWS, c.shape[2]), jnp.float32),
            pltpu.SemaphoreType.DMA((2,)),
        ],
        compiler_params=pltpu.CompilerParams(
            dimension_semantics=("arbitrary",),
        ),
    )(c, W, b2, x[..., -1].reshape(grid, 1, _ROWS), seg)
    return out.reshape(nb)


# speedup vs baseline: 1.2245x; 1.2245x over previous
import math

import jax
import jax.numpy as jnp
from jax import lax
from jax.experimental import pallas as pl
from jax.experimental.pallas import tpu as pltpu

_NB = 32
_NOUT = 3 * _NB + 1
_BOUND = 10.0
_MIN_W = 1e-05
_MIN_H = 1e-05
_MIN_D = 1e-05
_L2PI = 0.5 * math.log(2.0 * math.pi)
_PART = 16
_ROWS = 4096


def _flow_block(c_ref, w_ref, b_ref, x_ref, s_ref, o_ref):
    rows = c_ref.shape[0]
    theta = lax.dot_general(
        w_ref[...], c_ref[...],
        dimension_numbers=(((0,), (1,)), ((), ())),
        preferred_element_type=jnp.float32,
    ) + b_ref[...]

    uw = theta[0:_NB, :]
    uh = theta[_NB:2 * _NB, :]
    ud = theta[2 * _NB:_NOUT, :]

    xrow = x_ref[0]
    inside = (xrow >= -_BOUND) & (xrow <= _BOUND)
    xq = jnp.clip(xrow, -_BOUND, _BOUND)

    rid = lax.broadcasted_iota(jnp.int32, (_NB, rows), 0)

    def edges(u, min_v):
        e = jnp.exp(u)
        ecum = e
        for k in (1, 2, 4, 8, 16):
            ecum = ecum + jnp.concatenate(
                [jnp.zeros((k, rows), jnp.float32), ecum[:_NB - k, :]], axis=0)
        scale = (2.0 * _BOUND) * (1.0 - min_v * _NB) / ecum[_NB - 1:_NB, :]
        base = (2.0 * _BOUND * min_v) * (rid + 1).astype(jnp.float32) - _BOUND
        cum = base + ecum * scale
        cum = jnp.where(rid == _NB - 1, _BOUND, cum)
        left = jnp.concatenate(
            [jnp.full((1, rows), -_BOUND, jnp.float32), cum[:_NB - 1, :]], axis=0)
        return left, cum - left, cum

    cwl, wb, cwr = edges(uw, _MIN_W)
    chl, hb, _ = edges(uh, _MIN_H)

    one = jnp.ones((1, rows), jnp.float32)
    d = jnp.concatenate(
        [one, _MIN_D + jax.nn.softplus(ud[1:_NB, :]), one], axis=0)

    g = (xq >= cwr[:_NB - 1, :]).astype(jnp.float32)
    onehot = (jnp.concatenate([jnp.ones((1, rows), jnp.float32), g], axis=0)
              - jnp.concatenate([g, jnp.zeros((1, rows), jnp.float32)], axis=0))

    def pick(a):
        return jnp.sum(onehot * a, axis=0, keepdims=True)

    in_cw = pick(cwl)
    in_w = pick(wb)
    in_ch = pick(chl)
    in_h = pick(hb)
    d0 = pick(d[0:_NB, :])
    d1 = pick(d[1:_NB + 1, :])

    t = (xq - in_cw) / in_w
    tm = t * (1.0 - t)
    delta = in_h / in_w
    num = in_h * (delta * t * t + d0 * tm)
    den = delta + (d0 + d1 - 2.0 * delta) * tm
    outv = in_ch + num / den
    dnum = (delta * delta) * (d1 * t * t + 2.0 * delta * tm
                              + d0 * (1.0 - t) * (1.0 - t))
    lad = jnp.log(dnum) - 2.0 * jnp.log(den)

    z = jnp.where(inside, outv, xrow)
    jac = jnp.where(inside, lad, 0.0)
    prob = -_L2PI - 0.5 * z * z + jac

    p_hi = prob.astype(jnp.bfloat16)
    p_lo = (prob - p_hi.astype(jnp.float32)).astype(jnp.bfloat16)
    p2 = jnp.concatenate([p_hi, p_lo], axis=0)
    dn = (((1,), (0,)), ((), ()))
    ps2 = lax.dot_general(p2, s_ref[...], dn, preferred_element_type=jnp.float32)
    psum = ps2[0:1, :] + ps2[1:2, :]
    o_ref[...] = psum.reshape(1, 1, rows // _PART)


def kernel(x, c, W, b):
    nb, npart, _ = x.shape
    n = nb * npart
    grid = n // _ROWS
    bb = _ROWS // _PART

    be = _ROWS // npart
    b2 = b.reshape(-1, 1)
    seg = (jnp.arange(_ROWS, dtype=jnp.int32)[:, None] // _PART
           == jnp.arange(bb, dtype=jnp.int32)[None, :]).astype(jnp.bfloat16)

    out = pl.pallas_call(
        _flow_block,
        grid=(grid,),
        in_specs=[
            pl.BlockSpec((_ROWS, c.shape[2]), lambda i: (i, 0)),
            pl.BlockSpec(W.shape, lambda i: (0, 0)),
            pl.BlockSpec(b2.shape, lambda i: (0, 0)),
            pl.BlockSpec((1, 1, _ROWS), lambda i: (i, 0, 0)),
            pl.BlockSpec((_ROWS, bb), lambda i: (0, 0)),
        ],
        out_specs=pl.BlockSpec((1, 1, bb), lambda i: (i, 0, 0)),
        out_shape=jax.ShapeDtypeStruct((grid, 1, bb), jnp.float32),
        compiler_params=pltpu.CompilerParams(
            dimension_semantics=("parallel",),
        ),
    )(c.reshape(n, -1), W, b2, x[..., -1].reshape(grid, 1, _ROWS), seg)
    return out.reshape(nb)
